# baseline (speedup 1.0000x reference)
import dataclasses
import functools

import jax
import jax.numpy as jnp
from jax import lax
from jax.experimental import pallas as pl
from jax.experimental.pallas import tpu as pltpu
from jax.experimental.pallas import tpu_sc as plsc

BATCH = 16384
DIM = 32
LANES = 16
ROWS_PER_LINE = 4
NUM_CORES = 2
NUM_SUBCORES = 16
NUM_WORKERS = NUM_CORES * NUM_SUBCORES
BPW = BATCH // NUM_WORKERS
CHUNK = 256

TC_BLOCK = 512


def _gather_body(users_hbm, items_hbm, ut_hbm, it_hbm, uout_hbm, iout_hbm,
                 uidx_v, iidx_v, uridx_v, iridx_v, ubuf, vbuf, sem_u, sem_i):
    wid = lax.axis_index("s") * NUM_CORES + lax.axis_index("c")
    base = wid * BPW

    pltpu.sync_copy(users_hbm.at[pl.ds(base, BPW)], uidx_v)
    pltpu.sync_copy(items_hbm.at[pl.ds(base, BPW)], iidx_v)

    @pl.loop(0, BPW, step=LANES)
    def _(i0):
        uridx_v[pl.ds(i0, LANES)] = lax.shift_right_logical(
            uidx_v[pl.ds(i0, LANES)], 2)
        iridx_v[pl.ds(i0, LANES)] = lax.shift_right_logical(
            iidx_v[pl.ds(i0, LANES)], 2)

    for c in range(BPW // CHUNK):
        off = c * CHUNK
        cu = pltpu.async_copy(
            ut_hbm.at[uridx_v.at[pl.ds(off, CHUNK)]], ubuf, sem_u)
        ci = pltpu.async_copy(
            it_hbm.at[iridx_v.at[pl.ds(off, CHUNK)]], vbuf, sem_i)
        cu.wait()
        ci.wait()
        pltpu.sync_copy(ubuf, uout_hbm.at[pl.ds(base + off, CHUNK), :])
        pltpu.sync_copy(vbuf, iout_hbm.at[pl.ds(base + off, CHUNK), :])


def _sc_gather(users, items, ut, it):
    mesh = plsc.VectorSubcoreMesh(core_axis_name="c", subcore_axis_name="s")
    cp = dataclasses.replace(
        pltpu.CompilerParams(),
        needs_layout_passes=False,
        use_tc_tiling_on_sc=True,
    )
    run = pl.kernel(
        _gather_body,
        out_type=(
            jax.ShapeDtypeStruct((BATCH, 128), jnp.float32),
            jax.ShapeDtypeStruct((BATCH, 128), jnp.float32),
        ),
        mesh=mesh,
        scratch_types=[
            pltpu.VMEM((BPW,), jnp.int32),
            pltpu.VMEM((BPW,), jnp.int32),
            pltpu.VMEM((BPW,), jnp.int32),
            pltpu.VMEM((BPW,), jnp.int32),
            pltpu.VMEM((CHUNK, 128), jnp.float32),
            pltpu.VMEM((CHUNK, 128), jnp.float32),
            pltpu.SemaphoreType.DMA,
            pltpu.SemaphoreType.DMA,
        ],
        compiler_params=cp,
    )
    return run(users, items, ut, it)


def _finish_body(u_ref, v_ref, gu_ref, gv_ref, out_ref):
    gu = gu_ref[...] % ROWS_PER_LINE
    gv = gv_ref[...] % ROWS_PER_LINE
    acc = jnp.zeros((TC_BLOCK, 1), jnp.float32)
    for g in range(ROWS_PER_LINE):
        um = (gu == g).astype(jnp.float32)
        for h in range(ROWS_PER_LINE):
            vm = (gv == h).astype(jnp.float32)
            dots = jnp.sum(u_ref[:, g * DIM:(g + 1) * DIM]
                           * v_ref[:, h * DIM:(h + 1) * DIM],
                           axis=1, keepdims=True)
            acc = acc + um * vm * dots
    out_ref[...] = 1.0 / (1.0 + jnp.exp(-acc))


def _tc_finish(urows, irows, users_col, items_col):
    grid = (BATCH // TC_BLOCK,)
    return pl.pallas_call(
        _finish_body,
        out_shape=jax.ShapeDtypeStruct((BATCH, 1), jnp.float32),
        grid=grid,
        in_specs=[
            pl.BlockSpec((TC_BLOCK, 128), lambda i: (i, 0)),
            pl.BlockSpec((TC_BLOCK, 128), lambda i: (i, 0)),
            pl.BlockSpec((TC_BLOCK, 1), lambda i: (i, 0)),
            pl.BlockSpec((TC_BLOCK, 1), lambda i: (i, 0)),
        ],
        out_specs=pl.BlockSpec((TC_BLOCK, 1), lambda i: (i, 0)),
    )(urows, irows, users_col, items_col)


@jax.jit
def kernel(users, items, user_table, item_table):
    ut = user_table.reshape(250000, 128)
    it = item_table.reshape(250000, 128)
    urows, irows = _sc_gather(users, items, ut, it)
    out = _tc_finish(urows, irows,
                     users.reshape(BATCH, 1), items.reshape(BATCH, 1))
    return out.reshape(BATCH)
